# baseline (speedup 1.0000x reference)
import functools

import jax
import jax.numpy as jnp
from jax import lax
from jax.experimental import pallas as pl
from jax.experimental.pallas import tpu as pltpu
from jax.experimental.pallas import tpu_sc as plsc

_G = 2
_V = 320
_D = 128
_N = 1024
_GV = _G * _V
_NC, _NS = 1, 16
_NW = _NC * _NS
_TPW = _N // _NW
_BLK = 256


def _proj_argmax_body(h_ref, w_ref, b_ref, idx0_ref, idx1_ref, acc_ref):
    i = pl.program_id(0)
    k = pl.program_id(1)
    hs = h_ref[...].reshape(_BLK, h_ref.shape[-1])
    partial = jnp.dot(hs, w_ref[...], preferred_element_type=jnp.float32)

    @pl.when(k == 0)
    def _():
        acc_ref[...] = partial + b_ref[...]

    @pl.when(k == 1)
    def _():
        logits = acc_ref[...] + partial
        iota = lax.broadcasted_iota(jnp.int32, (_BLK, _V), 1)
        for g, out_ref in ((0, idx0_ref), (1, idx1_ref)):
            lg = logits[:, g * _V:(g + 1) * _V]
            m = jnp.max(lg, axis=1, keepdims=True)
            out_ref[pl.ds(i * _BLK, _BLK)] = (
                jnp.min(jnp.where(lg == m, iota, _V), axis=1) + g * _V)


def _ppl_body(idx0_ref, idx1_ref, ppl_ref):
    iota = lax.broadcasted_iota(jnp.int32, (_N, _V), 1)
    ppl = jnp.float32(0.0)
    for g, idx_ref in ((0, idx0_ref), (1, idx1_ref)):
        onehot = (iota == (idx_ref[...] - g * _V)[:, None]).astype(jnp.float32)
        p = jnp.sum(onehot, axis=0) * (1.0 / _N)
        ppl = ppl + jnp.exp(-jnp.sum(p * jnp.log(p + 1e-7)))
    ppl_ref[...] = jnp.broadcast_to(ppl, (1, 1))


@functools.cache
def _make_sc_gather():
    @functools.partial(
        pl.kernel,
        mesh=plsc.VectorSubcoreMesh(core_axis_name="c", subcore_axis_name="s",
                                    num_cores=_NC),
        out_type=jax.ShapeDtypeStruct((_N, _G * _D), jnp.float32),
        scratch_types=[
            pltpu.VMEM((_G * _TPW,), jnp.int32),
            pltpu.VMEM((_G * _TPW, _D), jnp.float32),
            pltpu.SemaphoreType.DMA,
            pltpu.SemaphoreType.DMA,
            pltpu.SemaphoreType.DMA,
        ],
    )
    def _sc_gather(table_hbm, idx0_hbm, idx1_hbm, out_hbm,
                   idx_v, rows_v, sem0, sem1, semg):
        wid = lax.axis_index("s") * _NC + lax.axis_index("c")
        base = wid * _TPW
        c0 = pltpu.async_copy(idx0_hbm.at[pl.ds(base, _TPW)],
                              idx_v.at[pl.ds(0, _TPW)], sem0)
        c1 = pltpu.async_copy(idx1_hbm.at[pl.ds(base, _TPW)],
                              idx_v.at[pl.ds(_TPW, _TPW)], sem1)
        c0.wait()
        c1.wait()
        pltpu.async_copy(table_hbm.at[idx_v], rows_v, semg).wait()
        w0 = pltpu.async_copy(rows_v.at[pl.ds(0, _TPW)],
                              out_hbm.at[pl.ds(base, _TPW), pl.ds(0, _D)], sem0)
        w1 = pltpu.async_copy(rows_v.at[pl.ds(_TPW, _TPW)],
                              out_hbm.at[pl.ds(base, _TPW), pl.ds(_D, _D)], sem1)
        w0.wait()
        w1.wait()

    return _sc_gather


def kernel(hidden_states, codevectors, W_proj, b_proj):
    b, s, h = hidden_states.shape
    nblk = _N // _BLK
    kh = h // 2
    idx0, idx1 = pl.pallas_call(
        _proj_argmax_body,
        grid=(nblk, 2),
        in_specs=[
            pl.BlockSpec((1, _BLK, kh), lambda i, k: (i, 0, k)),
            pl.BlockSpec((kh, _GV), lambda i, k: (k, 0)),
            pl.BlockSpec((_GV,), lambda i, k: (0,)),
        ],
        out_specs=(
            pl.BlockSpec((_N,), lambda i, k: (0,)),
            pl.BlockSpec((_N,), lambda i, k: (0,)),
        ),
        out_shape=(
            jax.ShapeDtypeStruct((_N,), jnp.int32),
            jax.ShapeDtypeStruct((_N,), jnp.int32),
        ),
        scratch_shapes=[pltpu.VMEM((_BLK, _GV), jnp.float32)],
    )(hidden_states, W_proj, b_proj)
    table = codevectors.reshape(_GV, _D)
    out = _make_sc_gather()(table, idx0, idx1)
    ppl = pl.pallas_call(
        _ppl_body,
        out_shape=jax.ShapeDtypeStruct((1, 1), jnp.float32),
    )(idx0, idx1)
    return (out.reshape(b, s, _G * _D), ppl[0, 0])


# speedup vs baseline: 1.1789x; 1.1789x over previous
import functools

import jax
import jax.numpy as jnp
from jax import lax
from jax.experimental import pallas as pl
from jax.experimental.pallas import tpu as pltpu
from jax.experimental.pallas import tpu_sc as plsc

_G = 2
_V = 320
_D = 128
_N = 1024
_GV = _G * _V
_NC, _NS = 1, 16
_NW = _NC * _NS
_TPW = _N // _NW
_BLK = 512


def _proj_argmax_body(h_ref, w_ref, b_ref, idx0_ref, idx1_ref):
    i = pl.program_id(0)
    hs = h_ref[...].reshape(_BLK, h_ref.shape[-1])
    logits = jnp.dot(hs, w_ref[...],
                     preferred_element_type=jnp.float32) + b_ref[...]
    iota = lax.broadcasted_iota(jnp.int32, (_BLK, _V), 1)
    for g, out_ref in ((0, idx0_ref), (1, idx1_ref)):
        lg = logits[:, g * _V:(g + 1) * _V]
        m = jnp.max(lg, axis=1, keepdims=True)
        out_ref[pl.ds(i * _BLK, _BLK)] = (
            jnp.min(jnp.where(lg == m, iota, _V), axis=1) + g * _V)


def _ppl_body(idx0_ref, idx1_ref, ppl_ref):
    iota = lax.broadcasted_iota(jnp.int32, (_N, _V), 1)
    ppl = jnp.float32(0.0)
    for g, idx_ref in ((0, idx0_ref), (1, idx1_ref)):
        onehot = (iota == (idx_ref[...] - g * _V)[:, None]).astype(jnp.float32)
        p = jnp.sum(onehot, axis=0) * (1.0 / _N)
        ppl = ppl + jnp.exp(-jnp.sum(p * jnp.log(p + 1e-7)))
    ppl_ref[...] = jnp.broadcast_to(ppl, (1, 1))


@functools.cache
def _make_sc_gather():
    @functools.partial(
        pl.kernel,
        mesh=plsc.VectorSubcoreMesh(core_axis_name="c", subcore_axis_name="s",
                                    num_cores=_NC),
        out_type=jax.ShapeDtypeStruct((_N, _G * _D), jnp.float32),
        scratch_types=[
            pltpu.VMEM((_G * _TPW,), jnp.int32),
            pltpu.VMEM((_G * _TPW, _D), jnp.float32),
            pltpu.SemaphoreType.DMA,
            pltpu.SemaphoreType.DMA,
            pltpu.SemaphoreType.DMA,
        ],
    )
    def _sc_gather(table_hbm, idx0_hbm, idx1_hbm, out_hbm,
                   idx_v, rows_v, sem0, sem1, semg):
        wid = lax.axis_index("s") * _NC + lax.axis_index("c")
        base = wid * _TPW
        c0 = pltpu.async_copy(idx0_hbm.at[pl.ds(base, _TPW)],
                              idx_v.at[pl.ds(0, _TPW)], sem0)
        c1 = pltpu.async_copy(idx1_hbm.at[pl.ds(base, _TPW)],
                              idx_v.at[pl.ds(_TPW, _TPW)], sem1)
        c0.wait()
        c1.wait()
        pltpu.async_copy(table_hbm.at[idx_v], rows_v, semg).wait()
        w0 = pltpu.async_copy(rows_v.at[pl.ds(0, _TPW)],
                              out_hbm.at[pl.ds(base, _TPW), pl.ds(0, _D)], sem0)
        w1 = pltpu.async_copy(rows_v.at[pl.ds(_TPW, _TPW)],
                              out_hbm.at[pl.ds(base, _TPW), pl.ds(_D, _D)], sem1)
        w0.wait()
        w1.wait()

    return _sc_gather


def kernel(hidden_states, codevectors, W_proj, b_proj):
    b, s, h = hidden_states.shape
    nblk = _N // _BLK
    idx0, idx1 = pl.pallas_call(
        _proj_argmax_body,
        grid=(nblk,),
        in_specs=[
            pl.BlockSpec((1, _BLK, h), lambda i: (i, 0, 0)),
            pl.BlockSpec((h, _GV), lambda i: (0, 0)),
            pl.BlockSpec((_GV,), lambda i: (0,)),
        ],
        out_specs=(
            pl.BlockSpec((_N,), lambda i: (0,)),
            pl.BlockSpec((_N,), lambda i: (0,)),
        ),
        out_shape=(
            jax.ShapeDtypeStruct((_N,), jnp.int32),
            jax.ShapeDtypeStruct((_N,), jnp.int32),
        ),
    )(hidden_states, W_proj, b_proj)
    table = codevectors.reshape(_GV, _D)
    out = _make_sc_gather()(table, idx0, idx1)
    ppl = pl.pallas_call(
        _ppl_body,
        out_shape=jax.ShapeDtypeStruct((1, 1), jnp.float32),
    )(idx0, idx1)
    return (out.reshape(b, s, _G * _D), ppl[0, 0])
